# baseline (speedup 1.0000x reference)
import jax
import jax.numpy as jnp
from jax.experimental import pallas as pl
from jax.experimental.pallas import tpu as pltpu


def _head_body(g_ref, xt_ref, wf1_ref, bf1_ref, wf2_ref, bf2_ref, wo_ref, bo_ref,
               out_ref):
    g = g_ref[...]
    xt = xt_ref[...]
    xc = jnp.concatenate([g, xt], axis=1)
    h1 = jnp.maximum(
        jax.lax.dot(xc, wf1_ref[...], preferred_element_type=jnp.float32)
        + bf1_ref[...][None, :], 0.0)
    h2 = jnp.maximum(
        jax.lax.dot(h1, wf2_ref[...], preferred_element_type=jnp.float32)
        + bf2_ref[...][None, :], 0.0)
    out_ref[...] = jax.lax.dot(h2, wo_ref[...],
                               preferred_element_type=jnp.float32) + bo_ref[...][None, :]


def _head(g, xt, Wf1, bf1, Wf2, bf2, Wo, bo):
    b = g.shape[0]
    return pl.pallas_call(
        _head_body,
        out_shape=jax.ShapeDtypeStruct((b, 1), jnp.float32),
    )(g, xt, Wf1, bf1, Wf2, bf2, Wo, bo)


def kernel(x, edge_index, batch, proteins, W1, att_src, att_dst, b1, W2, b2,
           Wg1, bg1, Wg2, bg2, emb, conv_w, conv_b, Wxt, bxt,
           Wf1, bf1, Wf2, bf2, Wo, bo):
    n = x.shape[0]
    b = proteins.shape[0]
    loops = jnp.arange(n, dtype=edge_index.dtype)
    src = jnp.concatenate([edge_index[0], loops])
    dst = jnp.concatenate([edge_index[1], loops])
    heads, d = att_src.shape
    h = (x @ W1).reshape(n, heads, d)
    a_src = jnp.sum(h * att_src[None, :, :], axis=-1)
    a_dst = jnp.sum(h * att_dst[None, :, :], axis=-1)
    e = a_src[src] + a_dst[dst]
    e = jnp.where(e > 0, e, 0.2 * e)
    e_max = jax.ops.segment_max(e, dst, num_segments=n)
    p = jnp.exp(e - e_max[dst])
    denom = jax.ops.segment_sum(p, dst, num_segments=n)
    alpha = p / (denom[dst] + 1e-16)
    x1 = jax.ops.segment_sum(h[src] * alpha[:, :, None], dst, num_segments=n)
    x1 = x1.reshape(n, heads * d) + b1
    x1 = jax.nn.relu(x1)
    deg = jax.ops.segment_sum(jnp.ones((src.shape[0],), dtype=x.dtype), dst,
                              num_segments=n)
    dinv = jax.lax.rsqrt(jnp.maximum(deg, 1.0))
    norm = dinv[src] * dinv[dst]
    h2 = x1 @ W2
    x2 = jax.ops.segment_sum(h2[src] * norm[:, None], dst, num_segments=n) + b2
    x2 = jax.nn.relu(x2)
    gmp = jax.ops.segment_max(x2, batch, num_segments=b)
    cnt = jax.ops.segment_sum(jnp.ones((n,), dtype=x.dtype), batch, num_segments=b)
    gap = jax.ops.segment_sum(x2, batch, num_segments=b) / jnp.maximum(cnt, 1.0)[:, None]
    g = jnp.concatenate([gmp, gap], axis=1)
    g = jax.nn.relu(g @ Wg1 + bg1)
    g = g @ Wg2 + bg2
    et = emb[proteins]
    conv = jax.lax.conv_general_dilated(et, conv_w, window_strides=(1,),
                                        padding='VALID',
                                        dimension_numbers=('NCH', 'OIH', 'NCH'))
    conv = conv + conv_b[None, :, None]
    xt = conv.reshape(b, 32 * 121)
    xt = xt @ Wxt + bxt
    return _head(g, xt, Wf1, bf1, Wf2, bf2, Wo, bo)


# speedup vs baseline: 9.4742x; 9.4742x over previous
import functools

import jax
import jax.numpy as jnp
from jax import lax
from jax.experimental import pallas as pl
from jax.experimental.pallas import tpu as pltpu
from jax.experimental.pallas import tpu_sc as plsc

N = 10000
NPAD = 10240
B = 256
D = 78
H = 10
HD = 780
CW = 80
EP = 172032
KB = 768
SUB = KB // 128
NT = 16
ROWS_T = NPAD // NT

_MESH = plsc.VectorSubcoreMesh(core_axis_name="c", subcore_axis_name="s")
_SC_PARAMS = pltpu.CompilerParams(use_tc_tiling_on_sc=False)


def _f32(*shape):
    return jax.ShapeDtypeStruct(shape, jnp.float32)


def _k1_body(x_ref, w1_ref, mcat_ref, h_ref, as_ref, ad_ref):
    x = x_ref[...]
    h = jnp.dot(x, w1_ref[...], preferred_element_type=jnp.float32)
    ab = jnp.dot(x, mcat_ref[...], preferred_element_type=jnp.float32)
    as_ref[...] = ab[:, 0:16]
    ad_ref[...] = ab[:, 16:32]
    pieces = [jnp.pad(h[:, 78 * k:78 * k + 78], ((0, 0), (0, 2))) for k in range(H)]
    h_ref[...] = jnp.stack(pieces, axis=0)


def _k1(x_pad, W1, M_cat):
    return pl.pallas_call(
        _k1_body,
        grid=(20,),
        in_specs=[
            pl.BlockSpec((512, 78), lambda i: (i, 0)),
            pl.BlockSpec((78, 780), lambda i: (0, 0)),
            pl.BlockSpec((78, 32), lambda i: (0, 0)),
        ],
        out_specs=[
            pl.BlockSpec((H, 512, CW), lambda i: (0, i, 0)),
            pl.BlockSpec((512, 16), lambda i: (i, 0)),
            pl.BlockSpec((512, 16), lambda i: (i, 0)),
        ],
        out_shape=[_f32(H, NPAD, CW), _f32(NPAD, 16), _f32(NPAD, 16)],
    )(x_pad, W1, M_cat)


def _k2_body(src_hbm, dst_hbm, asrc_hbm, adst_hbm, p_hbm, dd_hbm,
             sbufs, dbufs, abuf_s, abuf_d, pbuf, acc, sem, sem2):
    c = lax.axis_index("c")
    s = lax.axis_index("s")
    epc = EP // 2
    ept = epc // NT
    nblk = ept // KB
    tile_base = c * epc + s * ept

    def zrow(i, _):
        pbuf[i, :] = jnp.zeros((16,), jnp.float32)
        return 0
    lax.fori_loop(0, KB, zrow, 0)
    pltpu.sync_copy(pbuf.at[pl.ds(0, ROWS_T)], acc.at[pl.ds(s * ROWS_T, ROWS_T)])
    plsc.subcore_barrier()

    def blk(bi, _):
        base = tile_base + bi * KB
        for j in range(SUB):
            pltpu.sync_copy(src_hbm.at[pl.ds(base + 128 * j, 128)], sbufs[j])
            pltpu.sync_copy(dst_hbm.at[pl.ds(base + 128 * j, 128)], dbufs[j])
        gs = [pltpu.async_copy(asrc_hbm.at[sbufs[j]],
                               abuf_s.at[pl.ds(128 * j, 128)], sem)
              for j in range(SUB)]
        gd = [pltpu.async_copy(adst_hbm.at[dbufs[j]],
                               abuf_d.at[pl.ds(128 * j, 128)], sem)
              for j in range(SUB)]
        for dsc in gs + gd:
            dsc.wait()

        def edge(e, _):
            ev = abuf_s[e, :] + abuf_d[e, :]
            ev = jnp.where(ev > 0, ev, 0.2 * ev)
            pbuf[e, :] = jnp.exp(ev)
            return 0
        lax.fori_loop(0, KB, edge, 0)
        pltpu.sync_copy(pbuf, p_hbm.at[pl.ds(base, KB)])
        sc = [pltpu.async_copy(pbuf.at[pl.ds(128 * j, 128)],
                               acc.at[dbufs[j]], sem2, add=True)
              for j in range(SUB)]
        for dsc in sc:
            dsc.wait()
        return 0
    lax.fori_loop(0, nblk, blk, 0)
    plsc.subcore_barrier()
    pltpu.sync_copy(acc.at[pl.ds(s * ROWS_T, ROWS_T)],
                    dd_hbm.at[c].at[pl.ds(s * ROWS_T, ROWS_T)])


def _k2(src_p, dst_p, a_src_t, a_dst_t):
    f = pl.kernel(
        _k2_body,
        out_type=(_f32(EP, 16), _f32(2, NPAD, 16)),
        mesh=_MESH,
        compiler_params=_SC_PARAMS,
        scratch_types=(
            [pltpu.VMEM((128,), jnp.int32) for _ in range(SUB)],
            [pltpu.VMEM((128,), jnp.int32) for _ in range(SUB)],
            pltpu.VMEM((KB, 16), jnp.float32),
            pltpu.VMEM((KB, 16), jnp.float32),
            pltpu.VMEM((KB, 16), jnp.float32),
            pltpu.VMEM_SHARED((NPAD, 16), jnp.float32),
            pltpu.SemaphoreType.DMA,
            pltpu.SemaphoreType.DMA,
        ),
    )
    return f(src_p, dst_p, a_src_t, a_dst_t)


def _k3_body(src_hbm, dst_hbm, p_hbm, h_hbm, x1_hbm,
             sbufs, dbufs, pcol, rows, acc, sem, sem2):
    c = lax.axis_index("c")
    s = lax.axis_index("s")
    ept = EP // NT
    nblk = ept // KB
    tile_base = s * ept

    for kl in range(5):
        k = 5 * c + kl

        def zrow(i, _):
            for q in range(5):
                rows[i, pl.ds(16 * q, 16)] = jnp.zeros((16,), jnp.float32)
            return 0
        lax.fori_loop(0, ROWS_T, zrow, 0)
        pltpu.sync_copy(rows.at[pl.ds(0, ROWS_T)],
                        acc.at[pl.ds(s * ROWS_T, ROWS_T)])
        plsc.subcore_barrier()

        def blk(bi, _):
            base = tile_base + bi * KB
            for j in range(SUB):
                pltpu.sync_copy(src_hbm.at[pl.ds(base + 128 * j, 128)], sbufs[j])
                pltpu.sync_copy(dst_hbm.at[pl.ds(base + 128 * j, 128)], dbufs[j])
            pltpu.sync_copy(p_hbm.at[k].at[pl.ds(base, KB)], pcol)
            gs = [pltpu.async_copy(h_hbm.at[k].at[sbufs[j]],
                                   rows.at[pl.ds(128 * j, 128)], sem)
                  for j in range(SUB)]
            for dsc in gs:
                dsc.wait()

            def mul(g, _):
                pv = pcol[pl.ds(16 * g, 16)]
                for j in range(16):
                    sp = jnp.broadcast_to(pv[j], (16,))
                    er = 16 * g + j
                    for q in range(5):
                        rows[er, pl.ds(16 * q, 16)] = (
                            rows[er, pl.ds(16 * q, 16)] * sp)
                return 0
            lax.fori_loop(0, KB // 16, mul, 0)
            sc = [pltpu.async_copy(rows.at[pl.ds(128 * j, 128)],
                                   acc.at[dbufs[j]], sem2, add=True)
                  for j in range(SUB)]
            for dsc in sc:
                dsc.wait()
            return 0
        lax.fori_loop(0, nblk, blk, 0)
        plsc.subcore_barrier()
        pltpu.sync_copy(acc.at[pl.ds(s * ROWS_T, ROWS_T)],
                        x1_hbm.at[k].at[pl.ds(s * ROWS_T, ROWS_T)])
        plsc.subcore_barrier()


def _k3(src_p, dst_p, p_t, h_pad):
    f = pl.kernel(
        _k3_body,
        out_type=_f32(H, NPAD, CW),
        mesh=_MESH,
        compiler_params=_SC_PARAMS,
        scratch_types=(
            [pltpu.VMEM((128,), jnp.int32) for _ in range(SUB)],
            [pltpu.VMEM((128,), jnp.int32) for _ in range(SUB)],
            pltpu.VMEM((KB,), jnp.float32),
            pltpu.VMEM((KB, CW), jnp.float32),
            pltpu.VMEM_SHARED((NPAD, CW), jnp.float32),
            pltpu.SemaphoreType.DMA,
            pltpu.SemaphoreType.DMA,
        ),
    )
    return f(src_p, dst_p, p_t, h_pad)


def _k4_body(src_hbm, dst_hbm, h_hbm, x2_hbm,
             sbufs, dbufs, rows, acc, sem, sem2):
    c = lax.axis_index("c")
    s = lax.axis_index("s")
    ept = EP // NT
    nblk = ept // KB
    tile_base = s * ept

    for kl in range(5):
        k = 5 * c + kl

        def zrow(i, _):
            for q in range(5):
                rows[i, pl.ds(16 * q, 16)] = jnp.zeros((16,), jnp.float32)
            return 0
        lax.fori_loop(0, ROWS_T, zrow, 0)
        pltpu.sync_copy(rows.at[pl.ds(0, ROWS_T)],
                        acc.at[pl.ds(s * ROWS_T, ROWS_T)])
        plsc.subcore_barrier()

        def blk(bi, _):
            base = tile_base + bi * KB
            for j in range(SUB):
                pltpu.sync_copy(src_hbm.at[pl.ds(base + 128 * j, 128)], sbufs[j])
                pltpu.sync_copy(dst_hbm.at[pl.ds(base + 128 * j, 128)], dbufs[j])
            gs = [pltpu.async_copy(h_hbm.at[k].at[sbufs[j]],
                                   rows.at[pl.ds(128 * j, 128)], sem)
                  for j in range(SUB)]
            for dsc in gs:
                dsc.wait()
            sc = [pltpu.async_copy(rows.at[pl.ds(128 * j, 128)],
                                   acc.at[dbufs[j]], sem2, add=True)
                  for j in range(SUB)]
            for dsc in sc:
                dsc.wait()
            return 0
        lax.fori_loop(0, nblk, blk, 0)
        plsc.subcore_barrier()
        pltpu.sync_copy(acc.at[pl.ds(s * ROWS_T, ROWS_T)],
                        x2_hbm.at[k].at[pl.ds(s * ROWS_T, ROWS_T)])
        plsc.subcore_barrier()


def _k4(src_p, dst_p, h2s_pad):
    f = pl.kernel(
        _k4_body,
        out_type=_f32(H, NPAD, CW),
        mesh=_MESH,
        compiler_params=_SC_PARAMS,
        scratch_types=(
            [pltpu.VMEM((128,), jnp.int32) for _ in range(SUB)],
            [pltpu.VMEM((128,), jnp.int32) for _ in range(SUB)],
            pltpu.VMEM((KB, CW), jnp.float32),
            pltpu.VMEM_SHARED((NPAD, CW), jnp.float32),
            pltpu.SemaphoreType.DMA,
            pltpu.SemaphoreType.DMA,
        ),
    )
    return f(src_p, dst_p, h2s_pad)


def _k5_body(x1a_ref, dd_ref, b1_ref, w2_ref, out_ref):
    dd = dd_ref[0] + dd_ref[1]
    deg = dd[:, 10:11]
    dinv = lax.rsqrt(jnp.maximum(deg, 1.0))
    pieces = []
    for k in range(H):
        den = dd[:, k:k + 1] + 1e-16
        pieces.append(x1a_ref[k, :, 0:78] / den)
    x1 = jnp.concatenate(pieces, axis=1) + b1_ref[...]
    x1 = jnp.maximum(x1, 0.0)
    h2 = jnp.dot(x1, w2_ref[...], preferred_element_type=jnp.float32)
    h2s = h2 * dinv
    out = [jnp.pad(h2s[:, 78 * k:78 * k + 78], ((0, 0), (0, 2))) for k in range(H)]
    out_ref[...] = jnp.stack(out, axis=0)


def _k5(x1_agg, dd_parts, b1, W2):
    return pl.pallas_call(
        _k5_body,
        grid=(20,),
        in_specs=[
            pl.BlockSpec((H, 512, CW), lambda i: (0, i, 0)),
            pl.BlockSpec((2, 512, 16), lambda i: (0, i, 0)),
            pl.BlockSpec((1, 780), lambda i: (0, 0)),
            pl.BlockSpec((780, 780), lambda i: (0, 0)),
        ],
        out_specs=pl.BlockSpec((H, 512, CW), lambda i: (0, i, 0)),
        out_shape=_f32(H, NPAD, CW),
    )(x1_agg, dd_parts, b1.reshape(1, 780), W2)


def _k5c_body(x2a_ref, dd_ref, b2_ref, out_ref):
    dd = dd_ref[0] + dd_ref[1]
    deg = dd[:, 10:11]
    dinv = lax.rsqrt(jnp.maximum(deg, 1.0))
    outs = []
    for k in range(H):
        v = x2a_ref[k, :, 0:78] * dinv + b2_ref[0, 78 * k:78 * k + 78]
        v = jnp.maximum(v, 0.0)
        outs.append(jnp.pad(v, ((0, 0), (0, 2))))
    out_ref[...] = jnp.stack(outs, axis=0)


def _k5c(x2_agg, dd_parts, b2):
    return pl.pallas_call(
        _k5c_body,
        grid=(20,),
        in_specs=[
            pl.BlockSpec((H, 512, CW), lambda i: (0, i, 0)),
            pl.BlockSpec((2, 512, 16), lambda i: (0, i, 0)),
            pl.BlockSpec((1, 780), lambda i: (0, 0)),
        ],
        out_specs=pl.BlockSpec((H, 512, CW), lambda i: (0, i, 0)),
        out_shape=_f32(H, NPAD, CW),
    )(x2_agg, dd_parts, b2.reshape(1, 780))


def _k6_body(x2_hbm, batch_hbm, gmp_hbm, gsum_hbm,
             xbuf, bbuf, pmax, psum, sem):
    c = lax.axis_index("c")
    s = lax.axis_index("s")
    pltpu.sync_copy(batch_hbm.at[pl.ds(s * ROWS_T, ROWS_T)], bbuf)
    plsc.subcore_barrier()
    for kl in range(5):
        k = 5 * c + kl

        def zrow(i, _):
            for q in range(5):
                pmax[i, pl.ds(16 * q, 16)] = jnp.zeros((16,), jnp.float32)
                psum[i, pl.ds(16 * q, 16)] = jnp.zeros((16,), jnp.float32)
            return 0
        lax.fori_loop(0, 272, zrow, 0)
        pltpu.async_copy(x2_hbm.at[k].at[pl.ds(s * ROWS_T, ROWS_T)], xbuf,
                         sem).wait()

        def node(gi, _):
            bv = bbuf[pl.ds(16 * gi, 16)]
            for j in range(16):
                g = bv[j]
                v = 16 * gi + j
                for q in range(5):
                    r = xbuf[v, pl.ds(16 * q, 16)]
                    m = pmax[g, pl.ds(16 * q, 16)]
                    pmax[g, pl.ds(16 * q, 16)] = jnp.maximum(m, r)
                    t = psum[g, pl.ds(16 * q, 16)]
                    psum[g, pl.ds(16 * q, 16)] = t + r
            return 0
        lax.fori_loop(0, ROWS_T // 16, node, 0)
        pltpu.sync_copy(pmax, gmp_hbm.at[k].at[s])
        pltpu.sync_copy(psum, gsum_hbm.at[k].at[s])


def _k6(x2_pad, batch_pad):
    f = pl.kernel(
        _k6_body,
        out_type=(_f32(H, NT, 272, CW), _f32(H, NT, 272, CW)),
        mesh=_MESH,
        compiler_params=_SC_PARAMS,
        scratch_types=(
            pltpu.VMEM((ROWS_T, CW), jnp.float32),
            pltpu.VMEM((ROWS_T,), jnp.int32),
            pltpu.VMEM((272, CW), jnp.float32),
            pltpu.VMEM((272, CW), jnp.float32),
            pltpu.SemaphoreType.DMA,
        ),
    )
    return f(x2_pad, batch_pad)


def _k7a_body(gmp_ref, gsum_ref, mx_ref, sm_ref):
    mx_ref[...] = jnp.max(gmp_ref[0, :, 0:256, :], axis=0)[None]
    sm_ref[...] = jnp.sum(gsum_ref[0, :, 0:256, :], axis=0)[None]


def _k7a(gmp_parts, gsum_parts):
    return pl.pallas_call(
        _k7a_body,
        grid=(H,),
        in_specs=[
            pl.BlockSpec((1, NT, 272, CW), lambda i: (i, 0, 0, 0)),
            pl.BlockSpec((1, NT, 272, CW), lambda i: (i, 0, 0, 0)),
        ],
        out_specs=[
            pl.BlockSpec((1, 256, CW), lambda i: (i, 0, 0)),
            pl.BlockSpec((1, 256, CW), lambda i: (i, 0, 0)),
        ],
        out_shape=[_f32(H, 256, CW), _f32(H, 256, CW)],
    )(gmp_parts, gsum_parts)


def _k7mlp_body(gmp_ref, gsum_ref, b2d_ref, wg1_ref, bg1_ref, wg2_ref, bg2_ref,
                out_ref):
    b2d = b2d_ref[...]
    gid = lax.broadcasted_iota(jnp.int32, (256, 80, 128), 0)
    cnt = jnp.sum((b2d[None, :, :] == gid).astype(jnp.float32), axis=(1, 2))
    gap = gsum_ref[...] / jnp.maximum(cnt, 1.0)[:, None]
    gcat = jnp.concatenate([gmp_ref[...], gap], axis=1)
    g1 = jnp.dot(gcat, wg1_ref[...], preferred_element_type=jnp.float32)
    g1 = jnp.maximum(g1 + bg1_ref[...], 0.0)
    out_ref[...] = jnp.dot(g1, wg2_ref[...],
                           preferred_element_type=jnp.float32) + bg2_ref[...]


def _k7mlp(gmp_t, gsum_t, batch2d, Wg1, bg1, Wg2, bg2):
    return pl.pallas_call(
        _k7mlp_body,
        out_shape=_f32(256, 128),
    )(gmp_t, gsum_t, batch2d, Wg1, bg1.reshape(1, 1500), Wg2, bg2.reshape(1, 128))


def _ku_body(p_ref, w2d_ref, u_ref):
    p = p_ref[...]
    w2d = w2d_ref[...]
    for l in range(26):
        m = (p == l).astype(jnp.float32)
        u_ref[:, l, :] = jnp.dot(m, w2d, preferred_element_type=jnp.float32)


def _ku(proteins, w2d):
    return pl.pallas_call(
        _ku_body,
        grid=(2,),
        in_specs=[
            pl.BlockSpec((128, 1000), lambda i: (i, 0)),
            pl.BlockSpec((1000, 256), lambda i: (0, 0)),
        ],
        out_specs=pl.BlockSpec((128, 26, 256), lambda i: (i, 0, 0)),
        out_shape=_f32(256, 26, 256),
    )(proteins, w2d)


def _kt_body(ut_ref, t_ref, o_ref):
    o_ref[...] = jnp.dot(ut_ref[...], t_ref[...],
                         preferred_element_type=jnp.float32)


def _kt(U_t, T):
    return pl.pallas_call(
        _kt_body,
        grid=(16,),
        in_specs=[
            pl.BlockSpec((512, 208), lambda i: (i, 0)),
            pl.BlockSpec((208, 121), lambda i: (0, 0)),
        ],
        out_specs=pl.BlockSpec((512, 121), lambda i: (i, 0)),
        out_shape=_f32(8192, 121),
    )(U_t, T)


def _khead_body(g2_ref, cr_ref, wxt_ref, bxt_ref, wf1_ref, bf1_ref,
                wf2_ref, bf2_ref, wo_ref, bo_ref, out_ref):
    xt = jnp.dot(cr_ref[...], wxt_ref[...],
                 preferred_element_type=jnp.float32) + bxt_ref[...]
    xc = jnp.concatenate([g2_ref[...], xt], axis=1)
    h1 = jnp.maximum(jnp.dot(xc, wf1_ref[...],
                             preferred_element_type=jnp.float32) + bf1_ref[...], 0.0)
    h2 = jnp.maximum(jnp.dot(h1, wf2_ref[...],
                             preferred_element_type=jnp.float32) + bf2_ref[...], 0.0)
    out_ref[...] = jnp.dot(h2, wo_ref[...],
                           preferred_element_type=jnp.float32) + bo_ref[...]


def _khead(g2, conv_r2, Wxt, bxt2, Wf1, bf1, Wf2, bf2, Wo, bo):
    return pl.pallas_call(
        _khead_body,
        out_shape=_f32(256, 1),
    )(g2, conv_r2, Wxt, bxt2.reshape(1, 128), Wf1, bf1.reshape(1, 1024),
      Wf2, bf2.reshape(1, 512), Wo, bo.reshape(1, 1))


def kernel(x, edge_index, batch, proteins, W1, att_src, att_dst, b1, W2, b2,
           Wg1, bg1, Wg2, bg2, emb, conv_w, conv_b, Wxt, bxt,
           Wf1, bf1, Wf2, bf2, Wo, bo):
    x_pad = jnp.pad(x, ((0, NPAD - N), (0, 0)))
    loops = jnp.arange(N, dtype=edge_index.dtype)
    npad_e = EP - (160000 + N)
    dummy = jnp.full((npad_e,), N, edge_index.dtype)
    src_p = jnp.concatenate([edge_index[0], loops, dummy])
    dst_p = jnp.concatenate([edge_index[1], loops, dummy])
    batch_pad = jnp.concatenate(
        [batch, jnp.full((NPAD - N,), 256, batch.dtype)])

    w1r = W1.reshape(D, H, D)
    m_src = jnp.einsum('jkd,kd->jk', w1r, att_src)
    m_dst = jnp.einsum('jkd,kd->jk', w1r, att_dst)
    zpad = jnp.zeros((D, 6), jnp.float32)
    M_cat = jnp.concatenate([m_src, zpad, m_dst, zpad], axis=1)

    w2d = conv_w.transpose(1, 0, 2).reshape(1000, 256)
    T = jnp.stack([emb[:, i:i + 121] for i in range(8)], axis=1).reshape(208, 121)
    bxt2 = bxt + jnp.einsum('o,ojm->m', conv_b, Wxt.reshape(32, 121, 128))
    batch2d = batch_pad.reshape(80, 128)

    U = _ku(proteins, w2d)
    U_t = U.reshape(256, 26, 32, 8).transpose(0, 2, 1, 3).reshape(8192, 208)
    conv_r = _kt(U_t, T)
    conv_r2 = conv_r.reshape(256, 3872)

    h_pad, a_src_t, a_dst_t = _k1(x_pad, W1, M_cat)
    p_rows, dd_parts = _k2(src_p, dst_p, a_src_t, a_dst_t)
    p_t = p_rows.T[0:H].reshape(H, EP)
    x1_agg = _k3(src_p, dst_p, p_t, h_pad)
    h2s_pad = _k5(x1_agg, dd_parts, b1, W2)
    x2_agg = _k4(src_p, dst_p, h2s_pad)
    x2_pad = _k5c(x2_agg, dd_parts, b2)
    gmp_parts, gsum_parts = _k6(x2_pad, batch_pad)
    gmp_c, gsum_c = _k7a(gmp_parts, gsum_parts)
    gmp_t = gmp_c.transpose(1, 0, 2)[:, :, 0:78].reshape(256, 780)
    gsum_t = gsum_c.transpose(1, 0, 2)[:, :, 0:78].reshape(256, 780)
    g2 = _k7mlp(gmp_t, gsum_t, batch2d, Wg1, bg1, Wg2, bg2)

    return _khead(g2, conv_r2, Wxt, bxt2, Wf1, bf1, Wf2, bf2, Wo, bo)
